# baseline (speedup 1.0000x reference)
import functools

import jax
import jax.numpy as jnp
from jax import lax
from jax.experimental import pallas as pl
from jax.experimental.pallas import tpu as pltpu
from jax.experimental.pallas import tpu_sc as plsc

_D = 64
_MAX_IN = 8
_NC, _NS = 2, 16
_NW = _NC * _NS
_STEP_IDX = 256
_ROWS_PER_STEP = _STEP_IDX // _MAX_IN
_NBUF = 6
_LANES = 16


def _gather_sum_sc(table, idx_grp):
    total_steps = idx_grp.shape[0]
    n_out = total_steps * _ROWS_PER_STEP
    out_dtype = jnp.bfloat16
    chunk_rows = _ROWS_PER_STEP
    per_sub = total_steps // _NS
    s0 = (per_sub * 13 + 10) // 20
    s1 = per_sub - s0
    smax = max(s0, s1)
    mesh = plsc.VectorSubcoreMesh(core_axis_name="c", subcore_axis_name="s")

    @functools.partial(
        pl.kernel,
        out_type=jax.ShapeDtypeStruct((n_out, _D), out_dtype),
        mesh=mesh,
        compiler_params=pltpu.CompilerParams(use_tc_tiling_on_sc=False,
                                             needs_layout_passes=False),
        scratch_types=[
            pltpu.VMEM((smax, _STEP_IDX), jnp.int32),
            pltpu.VMEM((_NBUF, _STEP_IDX, _D), jnp.bfloat16),
            pltpu.VMEM((2, chunk_rows, _D), jnp.bfloat16),
            pltpu.SemaphoreType.DMA,
            pltpu.SemaphoreType.DMA,
            pltpu.SemaphoreType.DMA,
        ],
    )
    def gather_kernel(table_hbm, idx_hbm, out_hbm, idx_v, gbuf, obuf,
                      gsem, osem, isem):
        cid = lax.axis_index("c")
        sid = lax.axis_index("s")

        def run(nsteps, start):
            row_base = start * _ROWS_PER_STEP
            pltpu.async_copy(
                idx_hbm.at[pl.ds(start, nsteps)],
                idx_v.at[pl.ds(0, nsteps)], isem).wait()
            for b in range(_NBUF):
                pltpu.async_copy(table_hbm.at[idx_v.at[b]], gbuf.at[b], gsem)

            def step_body(i, carry):
                b = lax.rem(i, _NBUF)
                p = lax.rem(i, 2)

                @pl.when(i >= 2)
                def _():
                    pltpu.make_async_copy(
                        obuf.at[0],
                        out_hbm.at[pl.ds(row_base, chunk_rows)],
                        osem).wait()

                pltpu.make_async_copy(
                    table_hbm.at[idx_v.at[i]], gbuf.at[b], gsem).wait()

                def row_body(r, c2):
                    for cc in range(_D // (2 * _LANES)):
                        col = pl.ds(cc * 2 * _LANES, 2 * _LANES)
                        acc0 = jnp.zeros((_LANES,), jnp.float32)
                        acc1 = jnp.zeros((_LANES,), jnp.float32)
                        for j in range(_MAX_IN):
                            u = plsc.bitcast(
                                gbuf[b, r * _MAX_IN + j, col], jnp.int32)
                            acc0 = acc0 + plsc.bitcast(
                                u << jnp.int32(16), jnp.float32)
                            acc1 = acc1 + plsc.bitcast(
                                u & jnp.int32(-65536), jnp.float32)
                        u0 = plsc.bitcast(acc0, jnp.int32)
                        r0 = u0 + jnp.int32(0x7FFF) + (
                            lax.shift_right_logical(u0, 16) & jnp.int32(1))
                        u1 = plsc.bitcast(acc1, jnp.int32)
                        r1 = u1 + jnp.int32(0x7FFF) + (
                            lax.shift_right_logical(u1, 16) & jnp.int32(1))
                        packed = (lax.shift_right_logical(r0, 16)
                                  | (r1 & jnp.int32(-65536)))
                        obuf[p, r, col] = plsc.bitcast(packed, jnp.bfloat16)
                    return c2

                lax.fori_loop(0, _ROWS_PER_STEP, row_body, 0, unroll=2)

                @pl.when(i + _NBUF < nsteps)
                def _():
                    pltpu.async_copy(
                        table_hbm.at[idx_v.at[i + _NBUF]], gbuf.at[b], gsem)

                pltpu.async_copy(
                    obuf.at[p],
                    out_hbm.at[pl.ds(row_base + i * chunk_rows, chunk_rows)],
                    osem)
                return carry

            lax.fori_loop(0, nsteps, step_body, 0)
            for _ in range(2):
                pltpu.make_async_copy(
                    obuf.at[0],
                    out_hbm.at[pl.ds(row_base, chunk_rows)],
                    osem).wait()

        @pl.when(cid == 0)
        def _():
            run(s0, sid * per_sub)

        @pl.when(cid == 1)
        def _():
            run(s1, sid * per_sub + s0)

    return gather_kernel(table, idx_grp)


def _mm_relu_tc(x_t, w):
    k, n = x_t.shape
    bn = 2048

    def body(x_ref, w_ref, inp_ref, msg_ref):
        acc = jax.lax.dot_general(
            x_ref[...], w_ref[...], (((0,), (0,)), ((), ())),
            preferred_element_type=jnp.float32)
        inp_ref[...] = acc
        msg_ref[...] = jnp.maximum(acc, 0.0).astype(jnp.bfloat16)

    return pl.pallas_call(
        body,
        grid=(pl.cdiv(n, bn),),
        in_specs=[pl.BlockSpec((k, bn), lambda i: (0, i)),
                  pl.BlockSpec((k, _D), lambda i: (0, 0))],
        out_specs=[pl.BlockSpec((bn, _D), lambda i: (i, 0)),
                   pl.BlockSpec((bn, _D), lambda i: (i, 0))],
        out_shape=[jax.ShapeDtypeStruct((n, _D), jnp.float32),
                   jax.ShapeDtypeStruct((n, _D), jnp.bfloat16)],
    )(x_t, w)


def _update_tc(inp, msum_pad, wh):
    n = inp.shape[0]
    bn = 2048

    def body(inp_ref, ms_ref, wh_ref, out_ref):
        ms = ms_ref[...].astype(jnp.float32)
        out_ref[...] = jnp.maximum(
            inp_ref[...]
            + jnp.dot(ms, wh_ref[...], preferred_element_type=jnp.float32),
            0.0).astype(jnp.bfloat16)

    return pl.pallas_call(
        body,
        grid=(pl.cdiv(n, bn),),
        in_specs=[pl.BlockSpec((bn, _D), lambda i: (i, 0)),
                  pl.BlockSpec((bn, _D), lambda i: (i, 0)),
                  pl.BlockSpec((_D, _D), lambda i: (0, 0))],
        out_specs=pl.BlockSpec((bn, _D), lambda i: (i, 0)),
        out_shape=jax.ShapeDtypeStruct((n, _D), jnp.bfloat16),
    )(inp, msum_pad, wh)


def _atom_tc(af, msg_a_pad, wo_a, wo_m):
    n, fa = af.shape
    bn = 2000

    def body(af_ref, ms_ref, wa_ref, wm_ref, out_ref):
        ms = ms_ref[...].astype(jnp.float32)
        out_ref[...] = jnp.maximum(
            jnp.dot(af_ref[...], wa_ref[...], preferred_element_type=jnp.float32)
            + jnp.dot(ms, wm_ref[...], preferred_element_type=jnp.float32),
            0.0)

    return pl.pallas_call(
        body,
        grid=(n // bn,),
        in_specs=[pl.BlockSpec((bn, fa), lambda i: (i, 0)),
                  pl.BlockSpec((bn, _D), lambda i: (i, 0)),
                  pl.BlockSpec((fa, _D), lambda i: (0, 0)),
                  pl.BlockSpec((_D, _D), lambda i: (0, 0))],
        out_specs=pl.BlockSpec((bn, _D), lambda i: (i, 0)),
        out_shape=jax.ShapeDtypeStruct((n, _D), jnp.float32),
    )(af, msg_a_pad, wo_a, wo_m)


def _mol_tc(hidden, inv, n_mols, chunk):
    n = hidden.shape[0]
    mrows = ((n_mols + 7) // 8) * 8

    def body(inv_ref, h_ref, out_ref):
        r = lax.broadcasted_iota(jnp.int32, (mrows, n), 0)
        c = lax.broadcasted_iota(jnp.int32, (mrows, n), 1)
        sel = jnp.where(c // chunk == r, inv_ref[0], 0.0)
        out_ref[...] = jnp.dot(sel, h_ref[...], preferred_element_type=jnp.float32)

    return pl.pallas_call(
        body,
        grid=(1,),
        in_specs=[pl.BlockSpec(memory_space=pltpu.SMEM),
                  pl.BlockSpec((n, _D), lambda i: (0, 0))],
        out_specs=pl.BlockSpec((mrows, _D), lambda i: (0, 0)),
        out_shape=jax.ShapeDtypeStruct((mrows, _D), jnp.float32),
    )(inv, hidden)


def _group_idx(idx, rows_pad):
    flat = jnp.pad(idx.reshape(-1), (0, (rows_pad - idx.shape[0]) * _MAX_IN))
    return flat.reshape(rows_pad * _MAX_IN // _STEP_IDX, _STEP_IDX)


def kernel(atom_features, f_ini_atoms_bonds, atom_to_incoming_bonds, mapping,
           global_features, molecules_unbatch_key, W_i, W_h, W_o):
    nb1 = f_ini_atoms_bonds.shape[0]
    na = atom_features.shape[0]
    fa = atom_features.shape[1]

    align = _NW * _ROWS_PER_STEP
    nbp = ((nb1 + align - 1) // align) * align
    nap = ((na + align - 1) // align) * align
    map_grp = _group_idx(mapping, nbp)
    a2b_grp = _group_idx(atom_to_incoming_bonds, nap)

    inp, msg = _mm_relu_tc(f_ini_atoms_bonds.T, W_i)
    for _ in range(2):
        msum_pad = _gather_sum_sc(msg, map_grp)
        msg = _update_tc(inp, msum_pad, W_h)

    msg_a_pad = _gather_sum_sc(msg, a2b_grp)
    hidden = _atom_tc(atom_features, msg_a_pad, W_o[:fa], W_o[fa:])

    n_mols = global_features.shape[0]
    chunk = na // n_mols
    inv = (1.0 / jnp.asarray(molecules_unbatch_key, jnp.float32)).reshape(1)
    molp = _mol_tc(hidden, inv, n_mols, chunk)
    return jnp.concatenate([molp[:n_mols], global_features], axis=1)


# speedup vs baseline: 1.1142x; 1.1142x over previous
import functools

import jax
import jax.numpy as jnp
import numpy as np
from jax import lax
from jax.experimental import pallas as pl
from jax.experimental.pallas import tpu as pltpu
from jax.experimental.pallas import tpu_sc as plsc

_D = 64
_MAX_IN = 8
_NC, _NS = 2, 16
_NW = _NC * _NS
_STEP_IDX = 256
_ROWS_PER_STEP = _STEP_IDX // _MAX_IN
_NBUF = 6
_LANES = 16


_SRC_COL = np.array(
    [32 * (c // 32) + (2 * (c % 32) if c % 32 < 16 else 2 * (c % 32 - 16) + 1)
     for c in range(_D)])


def _gather_sum_sc(table, idx_grp):
    total_steps = idx_grp.shape[0]
    n_out = total_steps * _ROWS_PER_STEP
    chunk_rows = _ROWS_PER_STEP
    per_sub = total_steps // _NS
    s0 = (per_sub * 13 + 10) // 20
    s1 = per_sub - s0
    smax = max(s0, s1)
    mesh = plsc.VectorSubcoreMesh(core_axis_name="c", subcore_axis_name="s")

    @functools.partial(
        pl.kernel,
        out_type=jax.ShapeDtypeStruct((n_out, _D), jnp.float32),
        mesh=mesh,
        compiler_params=pltpu.CompilerParams(use_tc_tiling_on_sc=False,
                                             needs_layout_passes=False),
        scratch_types=[
            pltpu.VMEM((smax, _STEP_IDX), jnp.int32),
            pltpu.VMEM((_NBUF, _STEP_IDX, _D), jnp.bfloat16),
            pltpu.VMEM((2, chunk_rows, _D), jnp.float32),
            pltpu.SemaphoreType.DMA,
            pltpu.SemaphoreType.DMA,
            pltpu.SemaphoreType.DMA,
        ],
    )
    def gather_kernel(table_hbm, idx_hbm, out_hbm, idx_v, gbuf, obuf,
                      gsem, osem, isem):
        cid = lax.axis_index("c")
        sid = lax.axis_index("s")

        def run(nsteps, start):
            row_base = start * _ROWS_PER_STEP
            pltpu.async_copy(
                idx_hbm.at[pl.ds(start, nsteps)],
                idx_v.at[pl.ds(0, nsteps)], isem).wait()
            for b in range(_NBUF):
                pltpu.async_copy(table_hbm.at[idx_v.at[b]], gbuf.at[b], gsem)

            def step_body(i, carry):
                b = lax.rem(i, _NBUF)
                p = lax.rem(i, 2)

                @pl.when(i >= 2)
                def _():
                    pltpu.make_async_copy(
                        obuf.at[0],
                        out_hbm.at[pl.ds(row_base, chunk_rows)],
                        osem).wait()

                pltpu.make_async_copy(
                    table_hbm.at[idx_v.at[i]], gbuf.at[b], gsem).wait()

                def row_body(r, c2):
                    for cc in range(_D // (2 * _LANES)):
                        col = pl.ds(cc * 2 * _LANES, 2 * _LANES)
                        acc0 = jnp.zeros((_LANES,), jnp.float32)
                        acc1 = jnp.zeros((_LANES,), jnp.float32)
                        for j in range(_MAX_IN):
                            u = plsc.bitcast(
                                gbuf[b, r * _MAX_IN + j, col], jnp.int32)
                            acc0 = acc0 + plsc.bitcast(
                                u << jnp.int32(16), jnp.float32)
                            acc1 = acc1 + plsc.bitcast(
                                u & jnp.int32(-65536), jnp.float32)
                        obuf[p, r, pl.ds(cc * 2 * _LANES, _LANES)] = acc0
                        obuf[p, r, pl.ds(cc * 2 * _LANES + _LANES, _LANES)] = acc1
                    return c2

                lax.fori_loop(0, _ROWS_PER_STEP, row_body, 0, unroll=2)

                @pl.when(i + _NBUF < nsteps)
                def _():
                    pltpu.async_copy(
                        table_hbm.at[idx_v.at[i + _NBUF]], gbuf.at[b], gsem)

                pltpu.async_copy(
                    obuf.at[p],
                    out_hbm.at[pl.ds(row_base + i * chunk_rows, chunk_rows)],
                    osem)
                return carry

            lax.fori_loop(0, nsteps, step_body, 0)
            for _ in range(2):
                pltpu.make_async_copy(
                    obuf.at[0],
                    out_hbm.at[pl.ds(row_base, chunk_rows)],
                    osem).wait()

        @pl.when(cid == 0)
        def _():
            run(s0, sid * per_sub)

        @pl.when(cid == 1)
        def _():
            run(s1, sid * per_sub + s0)

    return gather_kernel(table, idx_grp)


def _mm_relu_tc(x_t, w):
    k, n = x_t.shape
    bn = 2048

    def body(x_ref, w_ref, inp_ref, msg_ref):
        acc = jax.lax.dot_general(
            x_ref[...], w_ref[...], (((0,), (0,)), ((), ())),
            preferred_element_type=jnp.float32)
        inp_ref[...] = acc
        msg_ref[...] = jnp.maximum(acc, 0.0).astype(jnp.bfloat16)

    return pl.pallas_call(
        body,
        grid=(pl.cdiv(n, bn),),
        in_specs=[pl.BlockSpec((k, bn), lambda i: (0, i)),
                  pl.BlockSpec((k, _D), lambda i: (0, 0))],
        out_specs=[pl.BlockSpec((bn, _D), lambda i: (i, 0)),
                   pl.BlockSpec((bn, _D), lambda i: (i, 0))],
        out_shape=[jax.ShapeDtypeStruct((n, _D), jnp.float32),
                   jax.ShapeDtypeStruct((n, _D), jnp.bfloat16)],
    )(x_t, w)


def _update_tc(inp, msum_pad, wh):
    n = inp.shape[0]
    bn = 2048

    def body(inp_ref, ms_ref, wh_ref, out_ref):
        out_ref[...] = jnp.maximum(
            inp_ref[...]
            + jnp.dot(ms_ref[...], wh_ref[...], preferred_element_type=jnp.float32),
            0.0).astype(jnp.bfloat16)

    return pl.pallas_call(
        body,
        grid=(pl.cdiv(n, bn),),
        in_specs=[pl.BlockSpec((bn, _D), lambda i: (i, 0)),
                  pl.BlockSpec((bn, _D), lambda i: (i, 0)),
                  pl.BlockSpec((_D, _D), lambda i: (0, 0))],
        out_specs=pl.BlockSpec((bn, _D), lambda i: (i, 0)),
        out_shape=jax.ShapeDtypeStruct((n, _D), jnp.bfloat16),
    )(inp, msum_pad, wh)


def _atom_tc(af, msg_a_pad, wo_a, wo_m):
    n, fa = af.shape
    bn = 2000

    def body(af_ref, ms_ref, wa_ref, wm_ref, out_ref):
        out_ref[...] = jnp.maximum(
            jnp.dot(af_ref[...], wa_ref[...], preferred_element_type=jnp.float32)
            + jnp.dot(ms_ref[...], wm_ref[...], preferred_element_type=jnp.float32),
            0.0)

    return pl.pallas_call(
        body,
        grid=(n // bn,),
        in_specs=[pl.BlockSpec((bn, fa), lambda i: (i, 0)),
                  pl.BlockSpec((bn, _D), lambda i: (i, 0)),
                  pl.BlockSpec((fa, _D), lambda i: (0, 0)),
                  pl.BlockSpec((_D, _D), lambda i: (0, 0))],
        out_specs=pl.BlockSpec((bn, _D), lambda i: (i, 0)),
        out_shape=jax.ShapeDtypeStruct((n, _D), jnp.float32),
    )(af, msg_a_pad, wo_a, wo_m)


def _mol_tc(hidden, inv, n_mols, chunk):
    n = hidden.shape[0]
    mrows = ((n_mols + 7) // 8) * 8

    def body(inv_ref, h_ref, out_ref):
        r = lax.broadcasted_iota(jnp.int32, (mrows, n), 0)
        c = lax.broadcasted_iota(jnp.int32, (mrows, n), 1)
        sel = jnp.where(c // chunk == r, inv_ref[0], 0.0)
        out_ref[...] = jnp.dot(sel, h_ref[...], preferred_element_type=jnp.float32)

    return pl.pallas_call(
        body,
        grid=(1,),
        in_specs=[pl.BlockSpec(memory_space=pltpu.SMEM),
                  pl.BlockSpec((n, _D), lambda i: (0, 0))],
        out_specs=pl.BlockSpec((mrows, _D), lambda i: (0, 0)),
        out_shape=jax.ShapeDtypeStruct((mrows, _D), jnp.float32),
    )(inv, hidden)


def _group_idx(idx, rows_pad):
    flat = jnp.pad(idx.reshape(-1), (0, (rows_pad - idx.shape[0]) * _MAX_IN))
    return flat.reshape(rows_pad * _MAX_IN // _STEP_IDX, _STEP_IDX)


def kernel(atom_features, f_ini_atoms_bonds, atom_to_incoming_bonds, mapping,
           global_features, molecules_unbatch_key, W_i, W_h, W_o):
    nb1 = f_ini_atoms_bonds.shape[0]
    na = atom_features.shape[0]
    fa = atom_features.shape[1]

    align = _NW * _ROWS_PER_STEP
    nbp = ((nb1 + align - 1) // align) * align
    nap = ((na + align - 1) // align) * align
    map_grp = _group_idx(mapping, nbp)
    a2b_grp = _group_idx(atom_to_incoming_bonds, nap)

    src = jnp.asarray(_SRC_COL)
    wh_perm = W_h[src]
    wo_m_perm = W_o[fa:][src]

    inp, msg = _mm_relu_tc(f_ini_atoms_bonds.T, W_i)
    for _ in range(2):
        msum_pad = _gather_sum_sc(msg, map_grp)
        msg = _update_tc(inp, msum_pad, wh_perm)

    msg_a_pad = _gather_sum_sc(msg, a2b_grp)
    hidden = _atom_tc(atom_features, msg_a_pad, W_o[:fa], wo_m_perm)

    n_mols = global_features.shape[0]
    chunk = na // n_mols
    inv = (1.0 / jnp.asarray(molecules_unbatch_key, jnp.float32)).reshape(1)
    molp = _mol_tc(hidden, inv, n_mols, chunk)
    return jnp.concatenate([molp[:n_mols], global_features], axis=1)


# speedup vs baseline: 1.1733x; 1.0531x over previous
import functools

import jax
import jax.numpy as jnp
import numpy as np
from jax import lax
from jax.experimental import pallas as pl
from jax.experimental.pallas import tpu as pltpu
from jax.experimental.pallas import tpu_sc as plsc

_D = 64
_MAX_IN = 8
_NC, _NS = 2, 16
_NW = _NC * _NS
_STEP_IDX = 256
_ROWS_PER_STEP = _STEP_IDX // _MAX_IN
_NBUF = 6
_LANES = 16


_SRC_COL = np.array(
    [32 * (c // 32) + (2 * (c % 32) if c % 32 < 16 else 2 * (c % 32 - 16) + 1)
     for c in range(_D)])


def _gather_sum_sc(table, idx_grp):
    total_steps = idx_grp.shape[0]
    n_out = total_steps * _ROWS_PER_STEP
    chunk_rows = _ROWS_PER_STEP
    per_sub = total_steps // _NS
    s0 = (per_sub * 13 + 10) // 20
    s1 = per_sub - s0
    smax = max(s0, s1)
    mesh = plsc.VectorSubcoreMesh(core_axis_name="c", subcore_axis_name="s")

    @functools.partial(
        pl.kernel,
        out_type=jax.ShapeDtypeStruct((n_out, _D), jnp.float32),
        mesh=mesh,
        compiler_params=pltpu.CompilerParams(use_tc_tiling_on_sc=False,
                                             needs_layout_passes=False),
        scratch_types=[
            pltpu.VMEM((smax, _STEP_IDX), jnp.int32),
            pltpu.VMEM((_NBUF, _STEP_IDX, _D), jnp.bfloat16),
            pltpu.VMEM((2, chunk_rows, _D), jnp.float32),
            pltpu.SemaphoreType.DMA,
            pltpu.SemaphoreType.DMA,
            pltpu.SemaphoreType.DMA,
        ],
    )
    def gather_kernel(table_hbm, idx_hbm, out_hbm, idx_v, gbuf, obuf,
                      gsem, osem, isem):
        cid = lax.axis_index("c")
        sid = lax.axis_index("s")

        def run(nsteps, start):
            row_base = start * _ROWS_PER_STEP
            pltpu.async_copy(
                idx_hbm.at[pl.ds(start, nsteps)],
                idx_v.at[pl.ds(0, nsteps)], isem).wait()
            for b in range(_NBUF):
                pltpu.async_copy(table_hbm.at[idx_v.at[b]], gbuf.at[b], gsem)

            def step_body(i, carry):
                b = lax.rem(i, _NBUF)
                p = lax.rem(i, 2)

                @pl.when(i >= 2)
                def _():
                    pltpu.make_async_copy(
                        obuf.at[0],
                        out_hbm.at[pl.ds(row_base, chunk_rows)],
                        osem).wait()

                pltpu.make_async_copy(
                    table_hbm.at[idx_v.at[i]], gbuf.at[b], gsem).wait()

                def row_body(r, c2):
                    for cc in range(_D // (2 * _LANES)):
                        col = pl.ds(cc * 2 * _LANES, 2 * _LANES)
                        acc0 = jnp.zeros((_LANES,), jnp.float32)
                        acc1 = jnp.zeros((_LANES,), jnp.float32)
                        for j in range(_MAX_IN):
                            u = plsc.bitcast(
                                gbuf[b, r * _MAX_IN + j, col], jnp.int32)
                            acc0 = acc0 + plsc.bitcast(
                                u << jnp.int32(16), jnp.float32)
                            acc1 = acc1 + plsc.bitcast(
                                u & jnp.int32(-65536), jnp.float32)
                        obuf[p, r, pl.ds(cc * 2 * _LANES, _LANES)] = acc0
                        obuf[p, r, pl.ds(cc * 2 * _LANES + _LANES, _LANES)] = acc1
                    return c2

                lax.fori_loop(0, _ROWS_PER_STEP, row_body, 0, unroll=2)

                @pl.when(i + _NBUF < nsteps)
                def _():
                    pltpu.async_copy(
                        table_hbm.at[idx_v.at[i + _NBUF]], gbuf.at[b], gsem)

                pltpu.async_copy(
                    obuf.at[p],
                    out_hbm.at[pl.ds(row_base + i * chunk_rows, chunk_rows)],
                    osem)
                return carry

            lax.fori_loop(0, nsteps, step_body, 0)
            for _ in range(2):
                pltpu.make_async_copy(
                    obuf.at[0],
                    out_hbm.at[pl.ds(row_base, chunk_rows)],
                    osem).wait()

        @pl.when(cid == 0)
        def _():
            run(s0, sid * per_sub)

        @pl.when(cid == 1)
        def _():
            run(s1, sid * per_sub + s0)

    return gather_kernel(table, idx_grp)


def _mm_relu_tc(x_t, w):
    k, n = x_t.shape
    bn = 4096

    def body(x_ref, w_ref, inp_ref, msg_ref):
        acc = jax.lax.dot_general(
            x_ref[...], w_ref[...], (((0,), (0,)), ((), ())),
            preferred_element_type=jnp.float32)
        inp_ref[...] = acc
        msg_ref[...] = jnp.maximum(acc, 0.0).astype(jnp.bfloat16)

    return pl.pallas_call(
        body,
        grid=(pl.cdiv(n, bn),),
        in_specs=[pl.BlockSpec((k, bn), lambda i: (0, i)),
                  pl.BlockSpec((k, _D), lambda i: (0, 0))],
        out_specs=[pl.BlockSpec((bn, _D), lambda i: (i, 0)),
                   pl.BlockSpec((bn, _D), lambda i: (i, 0))],
        out_shape=[jax.ShapeDtypeStruct((n, _D), jnp.float32),
                   jax.ShapeDtypeStruct((n, _D), jnp.bfloat16)],
    )(x_t, w)


def _update_tc(inp, msum_pad, wh):
    n = inp.shape[0]
    bn = 4096

    def body(inp_ref, ms_ref, wh_ref, out_ref):
        out_ref[...] = jnp.maximum(
            inp_ref[...]
            + jnp.dot(ms_ref[...], wh_ref[...], preferred_element_type=jnp.float32),
            0.0).astype(jnp.bfloat16)

    return pl.pallas_call(
        body,
        grid=(pl.cdiv(n, bn),),
        in_specs=[pl.BlockSpec((bn, _D), lambda i: (i, 0)),
                  pl.BlockSpec((bn, _D), lambda i: (i, 0)),
                  pl.BlockSpec((_D, _D), lambda i: (0, 0))],
        out_specs=pl.BlockSpec((bn, _D), lambda i: (i, 0)),
        out_shape=jax.ShapeDtypeStruct((n, _D), jnp.bfloat16),
    )(inp, msum_pad, wh)


def _atom_tc(af, msg_a_pad, wo_a, wo_m):
    n, fa = af.shape
    bn = 2000

    def body(af_ref, ms_ref, wa_ref, wm_ref, out_ref):
        out_ref[...] = jnp.maximum(
            jnp.dot(af_ref[...], wa_ref[...], preferred_element_type=jnp.float32)
            + jnp.dot(ms_ref[...], wm_ref[...], preferred_element_type=jnp.float32),
            0.0)

    return pl.pallas_call(
        body,
        grid=(n // bn,),
        in_specs=[pl.BlockSpec((bn, fa), lambda i: (i, 0)),
                  pl.BlockSpec((bn, _D), lambda i: (i, 0)),
                  pl.BlockSpec((fa, _D), lambda i: (0, 0)),
                  pl.BlockSpec((_D, _D), lambda i: (0, 0))],
        out_specs=pl.BlockSpec((bn, _D), lambda i: (i, 0)),
        out_shape=jax.ShapeDtypeStruct((n, _D), jnp.float32),
    )(af, msg_a_pad, wo_a, wo_m)


def _mol_tc(hidden, inv, n_mols, chunk):
    n = hidden.shape[0]
    mrows = ((n_mols + 7) // 8) * 8

    def body(inv_ref, h_ref, out_ref):
        r = lax.broadcasted_iota(jnp.int32, (mrows, n), 0)
        c = lax.broadcasted_iota(jnp.int32, (mrows, n), 1)
        sel = jnp.where(c // chunk == r, inv_ref[0], 0.0)
        out_ref[...] = jnp.dot(sel, h_ref[...], preferred_element_type=jnp.float32)

    return pl.pallas_call(
        body,
        grid=(1,),
        in_specs=[pl.BlockSpec(memory_space=pltpu.SMEM),
                  pl.BlockSpec((n, _D), lambda i: (0, 0))],
        out_specs=pl.BlockSpec((mrows, _D), lambda i: (0, 0)),
        out_shape=jax.ShapeDtypeStruct((mrows, _D), jnp.float32),
    )(inv, hidden)


def _group_idx(idx, rows_pad):
    flat = jnp.pad(idx.reshape(-1), (0, (rows_pad - idx.shape[0]) * _MAX_IN))
    return flat.reshape(rows_pad * _MAX_IN // _STEP_IDX, _STEP_IDX)


def kernel(atom_features, f_ini_atoms_bonds, atom_to_incoming_bonds, mapping,
           global_features, molecules_unbatch_key, W_i, W_h, W_o):
    nb1 = f_ini_atoms_bonds.shape[0]
    na = atom_features.shape[0]
    fa = atom_features.shape[1]

    align = _NW * _ROWS_PER_STEP
    nbp = ((nb1 + align - 1) // align) * align
    nap = ((na + align - 1) // align) * align
    map_grp = _group_idx(mapping, nbp)
    a2b_grp = _group_idx(atom_to_incoming_bonds, nap)

    src = jnp.asarray(_SRC_COL)
    wh_perm = W_h[src]
    wo_m_perm = W_o[fa:][src]

    inp, msg = _mm_relu_tc(f_ini_atoms_bonds.T, W_i)
    for _ in range(2):
        msum_pad = _gather_sum_sc(msg, map_grp)
        msg = _update_tc(inp, msum_pad, wh_perm)

    msg_a_pad = _gather_sum_sc(msg, a2b_grp)
    hidden = _atom_tc(atom_features, msg_a_pad, W_o[:fa], wo_m_perm)

    n_mols = global_features.shape[0]
    chunk = na // n_mols
    inv = (1.0 / jnp.asarray(molecules_unbatch_key, jnp.float32)).reshape(1)
    molp = _mol_tc(hidden, inv, n_mols, chunk)
    return jnp.concatenate([molp[:n_mols], global_features], axis=1)


# speedup vs baseline: 1.1903x; 1.0144x over previous
import functools

import jax
import jax.numpy as jnp
import numpy as np
from jax import lax
from jax.experimental import pallas as pl
from jax.experimental.pallas import tpu as pltpu
from jax.experimental.pallas import tpu_sc as plsc

_D = 64
_MAX_IN = 8
_NC, _NS = 2, 16
_NW = _NC * _NS
_STEP_IDX = 256
_ROWS_PER_STEP = _STEP_IDX // _MAX_IN
_NBUF = 6
_LANES = 16


_SRC_COL = np.array(
    [32 * (c // 32) + (2 * (c % 32) if c % 32 < 16 else 2 * (c % 32 - 16) + 1)
     for c in range(_D)])


def _gather_sum_sc(table, idx_grp):
    total_steps = idx_grp.shape[0]
    n_out = total_steps * _ROWS_PER_STEP
    chunk_rows = _ROWS_PER_STEP
    per_sub = total_steps // _NS
    s0 = (per_sub * 13 + 10) // 20
    s1 = per_sub - s0
    smax = max(s0, s1)
    mesh = plsc.VectorSubcoreMesh(core_axis_name="c", subcore_axis_name="s")

    @functools.partial(
        pl.kernel,
        out_type=jax.ShapeDtypeStruct((n_out, _D), jnp.float32),
        mesh=mesh,
        compiler_params=pltpu.CompilerParams(use_tc_tiling_on_sc=False,
                                             needs_layout_passes=False),
        scratch_types=[
            pltpu.VMEM((smax, _STEP_IDX), jnp.int32),
            pltpu.VMEM((_NBUF, _STEP_IDX, _D), jnp.bfloat16),
            pltpu.VMEM((2, chunk_rows, _D), jnp.float32),
            pltpu.SemaphoreType.DMA,
            pltpu.SemaphoreType.DMA,
            pltpu.SemaphoreType.DMA,
        ],
    )
    def gather_kernel(table_hbm, idx_hbm, out_hbm, idx_v, gbuf, obuf,
                      gsem, osem, isem):
        cid = lax.axis_index("c")
        sid = lax.axis_index("s")

        def run(nsteps, start):
            row_base = start * _ROWS_PER_STEP
            pltpu.async_copy(
                idx_hbm.at[pl.ds(start, nsteps)],
                idx_v.at[pl.ds(0, nsteps)], isem).wait()
            for b in range(_NBUF):
                pltpu.async_copy(table_hbm.at[idx_v.at[b]], gbuf.at[b], gsem)

            def step_body(i, carry):
                b = lax.rem(i, _NBUF)
                p = lax.rem(i, 2)

                @pl.when(i >= 2)
                def _():
                    pltpu.make_async_copy(
                        obuf.at[0],
                        out_hbm.at[pl.ds(row_base, chunk_rows)],
                        osem).wait()

                pltpu.make_async_copy(
                    table_hbm.at[idx_v.at[i]], gbuf.at[b], gsem).wait()

                def row_body(r, c2):
                    for cc in range(_D // (2 * _LANES)):
                        col = pl.ds(cc * 2 * _LANES, 2 * _LANES)
                        acc0 = jnp.zeros((_LANES,), jnp.float32)
                        acc1 = jnp.zeros((_LANES,), jnp.float32)
                        for j in range(_MAX_IN):
                            u = plsc.bitcast(
                                gbuf[b, r * _MAX_IN + j, col], jnp.int32)
                            acc0 = acc0 + plsc.bitcast(
                                u << jnp.int32(16), jnp.float32)
                            acc1 = acc1 + plsc.bitcast(
                                u & jnp.int32(-65536), jnp.float32)
                        obuf[p, r, pl.ds(cc * 2 * _LANES, _LANES)] = acc0
                        obuf[p, r, pl.ds(cc * 2 * _LANES + _LANES, _LANES)] = acc1
                    return c2

                lax.fori_loop(0, _ROWS_PER_STEP, row_body, 0, unroll=2)

                @pl.when(i + _NBUF < nsteps)
                def _():
                    pltpu.async_copy(
                        table_hbm.at[idx_v.at[i + _NBUF]], gbuf.at[b], gsem)

                pltpu.async_copy(
                    obuf.at[p],
                    out_hbm.at[pl.ds(row_base + i * chunk_rows, chunk_rows)],
                    osem)
                return carry

            lax.fori_loop(0, nsteps, step_body, 0)
            for _ in range(2):
                pltpu.make_async_copy(
                    obuf.at[0],
                    out_hbm.at[pl.ds(row_base, chunk_rows)],
                    osem).wait()

        @pl.when(cid == 0)
        def _():
            run(s0, sid * per_sub)

        @pl.when(cid == 1)
        def _():
            run(s1, sid * per_sub + s0)

    return gather_kernel(table, idx_grp)


def _mm_relu_tc(x_t, w):
    k, n = x_t.shape
    bn = 8192

    def body(x_ref, w_ref, inp_ref, msg_ref):
        acc = jax.lax.dot_general(
            x_ref[...], w_ref[...], (((0,), (0,)), ((), ())),
            preferred_element_type=jnp.float32)
        inp_ref[...] = acc
        msg_ref[...] = jnp.maximum(acc, 0.0).astype(jnp.bfloat16)

    return pl.pallas_call(
        body,
        grid=(pl.cdiv(n, bn),),
        in_specs=[pl.BlockSpec((k, bn), lambda i: (0, i)),
                  pl.BlockSpec((k, _D), lambda i: (0, 0))],
        out_specs=[pl.BlockSpec((bn, _D), lambda i: (i, 0)),
                   pl.BlockSpec((bn, _D), lambda i: (i, 0))],
        out_shape=[jax.ShapeDtypeStruct((n, _D), jnp.float32),
                   jax.ShapeDtypeStruct((n, _D), jnp.bfloat16)],
    )(x_t, w)


def _update_tc(inp, msum_pad, wh):
    n = inp.shape[0]
    bn = 8192

    def body(inp_ref, ms_ref, wh_ref, out_ref):
        out_ref[...] = jnp.maximum(
            inp_ref[...]
            + jnp.dot(ms_ref[...], wh_ref[...], preferred_element_type=jnp.float32),
            0.0).astype(jnp.bfloat16)

    return pl.pallas_call(
        body,
        grid=(pl.cdiv(n, bn),),
        in_specs=[pl.BlockSpec((bn, _D), lambda i: (i, 0)),
                  pl.BlockSpec((bn, _D), lambda i: (i, 0)),
                  pl.BlockSpec((_D, _D), lambda i: (0, 0))],
        out_specs=pl.BlockSpec((bn, _D), lambda i: (i, 0)),
        out_shape=jax.ShapeDtypeStruct((n, _D), jnp.bfloat16),
    )(inp, msum_pad, wh)


def _atom_tc(af, msg_a_pad, wo_a, wo_m):
    n, fa = af.shape
    bn = 2000

    def body(af_ref, ms_ref, wa_ref, wm_ref, out_ref):
        out_ref[...] = jnp.maximum(
            jnp.dot(af_ref[...], wa_ref[...], preferred_element_type=jnp.float32)
            + jnp.dot(ms_ref[...], wm_ref[...], preferred_element_type=jnp.float32),
            0.0)

    return pl.pallas_call(
        body,
        grid=(n // bn,),
        in_specs=[pl.BlockSpec((bn, fa), lambda i: (i, 0)),
                  pl.BlockSpec((bn, _D), lambda i: (i, 0)),
                  pl.BlockSpec((fa, _D), lambda i: (0, 0)),
                  pl.BlockSpec((_D, _D), lambda i: (0, 0))],
        out_specs=pl.BlockSpec((bn, _D), lambda i: (i, 0)),
        out_shape=jax.ShapeDtypeStruct((n, _D), jnp.float32),
    )(af, msg_a_pad, wo_a, wo_m)


def _mol_tc(hidden, inv, n_mols, chunk):
    n = hidden.shape[0]
    mrows = ((n_mols + 7) // 8) * 8

    def body(inv_ref, h_ref, out_ref):
        r = lax.broadcasted_iota(jnp.int32, (mrows, n), 0)
        c = lax.broadcasted_iota(jnp.int32, (mrows, n), 1)
        sel = jnp.where(c // chunk == r, inv_ref[0], 0.0)
        out_ref[...] = jnp.dot(sel, h_ref[...], preferred_element_type=jnp.float32)

    return pl.pallas_call(
        body,
        grid=(1,),
        in_specs=[pl.BlockSpec(memory_space=pltpu.SMEM),
                  pl.BlockSpec((n, _D), lambda i: (0, 0))],
        out_specs=pl.BlockSpec((mrows, _D), lambda i: (0, 0)),
        out_shape=jax.ShapeDtypeStruct((mrows, _D), jnp.float32),
    )(inv, hidden)


def _group_idx(idx, rows_pad):
    flat = jnp.pad(idx.reshape(-1), (0, (rows_pad - idx.shape[0]) * _MAX_IN))
    return flat.reshape(rows_pad * _MAX_IN // _STEP_IDX, _STEP_IDX)


def kernel(atom_features, f_ini_atoms_bonds, atom_to_incoming_bonds, mapping,
           global_features, molecules_unbatch_key, W_i, W_h, W_o):
    nb1 = f_ini_atoms_bonds.shape[0]
    na = atom_features.shape[0]
    fa = atom_features.shape[1]

    align = _NW * _ROWS_PER_STEP
    nbp = ((nb1 + align - 1) // align) * align
    nap = ((na + align - 1) // align) * align
    map_grp = _group_idx(mapping, nbp)
    a2b_grp = _group_idx(atom_to_incoming_bonds, nap)

    src = jnp.asarray(_SRC_COL)
    wh_perm = W_h[src]
    wo_m_perm = W_o[fa:][src]

    inp, msg = _mm_relu_tc(f_ini_atoms_bonds.T, W_i)
    for _ in range(2):
        msum_pad = _gather_sum_sc(msg, map_grp)
        msg = _update_tc(inp, msum_pad, wh_perm)

    msg_a_pad = _gather_sum_sc(msg, a2b_grp)
    hidden = _atom_tc(atom_features, msg_a_pad, W_o[:fa], wo_m_perm)

    n_mols = global_features.shape[0]
    chunk = na // n_mols
    inv = (1.0 / jnp.asarray(molecules_unbatch_key, jnp.float32)).reshape(1)
    molp = _mol_tc(hidden, inv, n_mols, chunk)
    return jnp.concatenate([molp[:n_mols], global_features], axis=1)
